# baseline (speedup 1.0000x reference)
import jax
import jax.numpy as jnp
from jax import lax
from jax.experimental import pallas as pl
from jax.experimental.pallas import tpu as pltpu


S, D, E = 2048, 768, 8


def _moe_step(x_ref, wgt_ref, bg_ref, w_ref, b_ref, o_ref, gates_s, xbf_s):
    e = pl.program_id(0)

    @pl.when(e == 0)
    def _():
        xb = x_ref[...]
        logits = jnp.dot(xb, wgt_ref[...], preferred_element_type=jnp.float32)
        logits = logits + bg_ref[...]

        iota = lax.broadcasted_iota(jnp.int32, (S, E), 1)
        i1 = jnp.argmax(logits, axis=-1)[:, None]
        one1 = iota == i1
        v1 = jnp.max(logits, axis=-1, keepdims=True)
        masked = jnp.where(one1, -jnp.inf, logits)
        i2 = jnp.argmax(masked, axis=-1)[:, None]
        one2 = iota == i2
        v2 = jnp.max(masked, axis=-1, keepdims=True)

        t = jnp.exp(v2 - v1)
        denom = 1.0 + t
        p1 = 1.0 / denom
        p2 = t / denom
        gates = jnp.where(one1, p1, 0.0) + jnp.where(one2, p2, 0.0)

        gates_s[...] = gates
        xbf_s[...] = xb.astype(jnp.bfloat16)
        o_ref[...] = jnp.dot(gates, b_ref[...], preferred_element_type=jnp.float32)

    ye = lax.dot_general(xbf_s[...], w_ref[0].astype(jnp.bfloat16),
                         (((1,), (1,)), ((), ())),
                         preferred_element_type=jnp.float32)
    g_all = gates_s[...]
    for e_s in range(E):
        @pl.when(e == e_s)
        def _(e_s=e_s):
            o_ref[...] = o_ref[...] + g_all[:, e_s][:, None] * ye


@jax.jit
def kernel(x, Wg, bg, W, b):
    x2 = x.reshape(S, D)
    WgT = Wg.T
    bg2 = bg.reshape(1, E)

    out = pl.pallas_call(
        _moe_step,
        grid=(E,),
        in_specs=[
            pl.BlockSpec((S, D), lambda e: (0, 0)),
            pl.BlockSpec((D, E), lambda e: (0, 0)),
            pl.BlockSpec((1, E), lambda e: (0, 0)),
            pl.BlockSpec((1, D, D), lambda e: (e, 0, 0)),
            pl.BlockSpec((E, D), lambda e: (0, 0)),
        ],
        out_specs=pl.BlockSpec((S, D), lambda e: (0, 0)),
        out_shape=jax.ShapeDtypeStruct((S, D), jnp.float32),
        scratch_shapes=[
            pltpu.VMEM((S, E), jnp.float32),
            pltpu.VMEM((S, D), jnp.bfloat16),
        ],
    )(x2, WgT, bg2, W, b)
    return out.reshape(1, S, D)


# speedup vs baseline: 1.2107x; 1.2107x over previous
import jax
import jax.numpy as jnp
from jax.experimental import pallas as pl


S, D, E = 2048, 768, 8
BS = 1024


def _moe_block(x_ref, wgt_ref, bg_ref, wt_ref, b_ref, o_ref):
    xb = x_ref[...]
    logits = jnp.dot(xb, wgt_ref[...], preferred_element_type=jnp.float32)
    logits = logits + bg_ref[...]

    iota = jax.lax.broadcasted_iota(jnp.int32, (BS, E), 1)
    i1 = jnp.argmax(logits, axis=-1)[:, None]
    one1 = iota == i1
    v1 = jnp.max(logits, axis=-1, keepdims=True)
    masked = jnp.where(one1, -jnp.inf, logits)
    i2 = jnp.argmax(masked, axis=-1)[:, None]
    one2 = iota == i2
    v2 = jnp.max(masked, axis=-1, keepdims=True)

    t = jnp.exp(v2 - v1)
    denom = 1.0 + t
    p1 = 1.0 / denom
    p2 = t / denom
    gates = jnp.where(one1, p1, 0.0) + jnp.where(one2, p2, 0.0)

    acc = jnp.dot(gates, b_ref[...], preferred_element_type=jnp.float32)
    xb_bf = xb.astype(jnp.bfloat16)
    for e in range(E):
        ye = jax.lax.dot_general(xb_bf, wt_ref[e].astype(jnp.bfloat16),
                                 (((1,), (1,)), ((), ())),
                                 preferred_element_type=jnp.float32)
        acc = acc + gates[:, e][:, None] * ye
    o_ref[...] = acc


@jax.jit
def kernel(x, Wg, bg, W, b):
    x2 = x.reshape(S, D)
    WgT = Wg.T
    bg2 = bg.reshape(1, E)

    out = pl.pallas_call(
        _moe_block,
        grid=(S // BS,),
        in_specs=[
            pl.BlockSpec((BS, D), lambda i: (i, 0)),
            pl.BlockSpec((D, E), lambda i: (0, 0)),
            pl.BlockSpec((1, E), lambda i: (0, 0)),
            pl.BlockSpec((E, D, D), lambda i: (0, 0, 0)),
            pl.BlockSpec((E, D), lambda i: (0, 0)),
        ],
        out_specs=pl.BlockSpec((BS, D), lambda i: (i, 0)),
        out_shape=jax.ShapeDtypeStruct((S, D), jnp.float32),
    )(x2, WgT, bg2, W, b)
    return out.reshape(1, S, D)
